# baseline (speedup 1.0000x reference)
import jax
import jax.numpy as jnp
from jax.experimental import pallas as pl

_GRID = 100
_M_BLK = 1000
_U_BLK = 10000


def _fused_kernel(x_ref, u_ref, w_ref, b_ref, uo_ref, mo_ref):
    uo_ref[...] = u_ref[...]
    acc = jnp.dot(x_ref[...], w_ref[...], preferred_element_type=jnp.float32)
    mo_ref[...] = jnp.maximum(acc + b_ref[...], 0.0)


def kernel(movie_x, user_emb_weight, W, b):
    n, f = movie_x.shape
    nu, e = user_emb_weight.shape
    user, movie = pl.pallas_call(
        _fused_kernel,
        grid=(_GRID,),
        in_specs=[
            pl.BlockSpec((_M_BLK, f), lambda i: (i, 0)),
            pl.BlockSpec((_U_BLK, e), lambda i: (i, 0)),
            pl.BlockSpec((f, e), lambda i: (0, 0)),
            pl.BlockSpec((1, e), lambda i: (0, 0)),
        ],
        out_specs=[
            pl.BlockSpec((_U_BLK, e), lambda i: (i, 0)),
            pl.BlockSpec((_M_BLK, e), lambda i: (i, 0)),
        ],
        out_shape=[
            jax.ShapeDtypeStruct((nu, e), jnp.float32),
            jax.ShapeDtypeStruct((n, e), jnp.float32),
        ],
    )(movie_x, user_emb_weight, W, b.reshape(1, -1))
    return (user, movie)
